# baseline (speedup 1.0000x reference)
import functools

import jax
import jax.numpy as jnp
from jax import lax
from jax.experimental import pallas as pl
from jax.experimental.pallas import tpu as pltpu
from jax.experimental.pallas import tpu_sc as plsc

VOCAB = 1000000
D = 64
B = 16384

NC = 2
NS = 16
NW = NC * NS
BPW = B // NW
QTR = BPW // 4
L = 16
UNROLL = 8

CV = 32768
GRID = (VOCAB + CV - 1) // CV
CVLOG = CV.bit_length() - 1

_SC_MESH = dict(core_axis_name="c", subcore_axis_name="s")


def _pack_tc(cwt_ref, swt_ref, p_ref):
    x = jnp.concatenate([cwt_ref[...], swt_ref[...]], axis=0)
    y = jax.lax.dot_general(
        x, jnp.eye(2 * D, dtype=jnp.float32), (((0,), (0,)), ((), ())),
        preferred_element_type=jnp.float32)
    yi = jax.lax.bitcast_convert_type(y, jnp.int32)
    rnd = jnp.int32(0x8000)
    hi = (yi[: CV // 2] + rnd) & jnp.int32(-65536)
    lo = jax.lax.shift_right_logical(yi[CV // 2:] + rnd, 16)
    p_ref[...] = hi | lo


def _pack_tables(cwt, swt):
    return pl.pallas_call(
        _pack_tc,
        grid=(GRID,),
        compiler_params=pltpu.CompilerParams(
            fuse_transposed_lhs_in_matmul=True),
        in_specs=[
            pl.BlockSpec((D, CV), lambda i: (0, i)),
            pl.BlockSpec((D, CV), lambda i: (0, i)),
        ],
        out_specs=pl.BlockSpec((CV // 2, 2 * D), lambda i: (i, 0)),
        out_shape=jax.ShapeDtypeStruct((GRID * CV // 2, 2 * D), jnp.int32),
    )(cwt, swt)


def _bias_sc(c_hbm, s_hbm, cb_hbm, sb_hbm, out_hbm,
             cidx_v, sidx_v, cb_v, sb_v, out_v, sem):
    wid = lax.axis_index("s") * NC + lax.axis_index("c")
    base = wid * BPW

    pltpu.sync_copy(c_hbm.at[pl.ds(base, BPW)], cidx_v)
    pltpu.sync_copy(s_hbm.at[pl.ds(base, BPW)], sidx_v)
    cp1 = pltpu.async_copy(cb_hbm.at[cidx_v], cb_v, sem)
    cp2 = pltpu.async_copy(sb_hbm.at[sidx_v], sb_v, sem)
    cp1.wait()
    cp2.wait()

    def body(g, carry):
        gs = g * L
        out_v[pl.ds(gs, L)] = cb_v[pl.ds(gs, L)] + sb_v[pl.ds(gs, L)]
        return carry

    lax.fori_loop(0, BPW // L, body, 0)
    pltpu.sync_copy(out_v, out_hbm.at[pl.ds(base, BPW)])


def _bias_sums(c, s, cb, sb):
    fn = functools.partial(
        pl.kernel,
        out_type=jax.ShapeDtypeStruct((B,), jnp.float32),
        mesh=plsc.VectorSubcoreMesh(**_SC_MESH),
        compiler_params=pltpu.CompilerParams(
            needs_layout_passes=False, use_tc_tiling_on_sc=False),
        scratch_types=[
            pltpu.VMEM((BPW,), jnp.int32),
            pltpu.VMEM((BPW,), jnp.int32),
            pltpu.VMEM((BPW,), jnp.float32),
            pltpu.VMEM((BPW,), jnp.float32),
            pltpu.VMEM((BPW,), jnp.float32),
            pltpu.SemaphoreType.DMA,
        ],
    )(_bias_sc)
    return fn(c, s, cb, sb)


def _glove_sc(c_hbm, s_hbm, m_hbm, bsum_hbm, out_hbm,
              cidx_v, sidx_v, cpair_v, spair_v, cbuf_v, sbuf_v, bsum_v,
              out_v, sem0, sem1):
    wid = lax.axis_index("s") * NC + lax.axis_index("c")
    base = wid * BPW

    pltpu.sync_copy(c_hbm.at[pl.ds(base, BPW)], cidx_v)
    pltpu.sync_copy(s_hbm.at[pl.ds(base, BPW)], sidx_v)
    pltpu.sync_copy(bsum_hbm.at[pl.ds(base, BPW)], bsum_v)

    def pair(v):
        return (lax.shift_left(lax.shift_right_logical(v, CVLOG), CVLOG - 1)
                | (v & (CV // 2 - 1)))

    def pair_body(g, carry):
        gs = g * L
        cpair_v[pl.ds(gs, L)] = pair(cidx_v[pl.ds(gs, L)])
        spair_v[pl.ds(gs, L)] = pair(sidx_v[pl.ds(gs, L)])
        return carry

    lax.fori_loop(0, BPW // L, pair_body, 0)

    nq = BPW // QTR

    def fire(q, buf_slot):
        qs = q * QTR
        sem = sem0 if buf_slot == 0 else sem1
        return (
            pltpu.async_copy(
                m_hbm.at[cpair_v.at[pl.ds(qs, QTR)]], cbuf_v.at[buf_slot],
                sem),
            pltpu.async_copy(
                m_hbm.at[spair_v.at[pl.ds(qs, QTR)]], sbuf_v.at[buf_slot],
                sem),
        )

    cps = {0: fire(0, 0)}
    for q in range(nq):
        if q + 1 < nq:
            cps[q + 1] = fire(q + 1, (q + 1) % 2)
        cp1, cp2 = cps.pop(q)
        cp1.wait()
        cp2.wait()
        qs = q * QTR
        slot = q % 2

        def group_body(g, carry2, qs=qs, slot=slot):
            gs = g * L
            row_idx = gs + lax.iota(jnp.int32, L)
            csh = (lax.shift_right_logical(
                cidx_v[pl.ds(qs + gs, L)], CVLOG - 1) & 1) << 4
            ssh = (lax.shift_right_logical(
                sidx_v[pl.ds(qs + gs, L)], CVLOG - 1) & 1) << 4
            mask = jnp.full((L,), -65536, jnp.int32)
            acc = bsum_v[pl.ds(qs + gs, L)]

            def d_body(d0, acc):
                for u in range(UNROLL):
                    d = d0 * UNROLL + u
                    col_c = jnp.full((L,), 0, jnp.int32) + d
                    col_s = col_c + D
                    wc = plsc.load_gather(cbuf_v.at[slot], [row_idx, col_c])
                    ws = plsc.load_gather(sbuf_v.at[slot], [row_idx, col_s])
                    cv = plsc.bitcast((wc << csh) & mask, jnp.float32)
                    sv = plsc.bitcast((ws << ssh) & mask, jnp.float32)
                    acc = acc + cv * sv
                return acc

            acc = lax.fori_loop(0, D // UNROLL, d_body, acc)
            out_v[pl.ds(qs + gs, L)] = acc
            return carry2

        lax.fori_loop(0, QTR // L, group_body, 0)

    pltpu.sync_copy(out_v, out_hbm.at[pl.ds(base, BPW)])


@jax.jit
def _glove(c, s, cwt, cb, swt, sb):
    bsum = _bias_sums(c, s, cb, sb)
    merged = _pack_tables(cwt, swt)
    fn = functools.partial(
        pl.kernel,
        out_type=jax.ShapeDtypeStruct((B,), jnp.float32),
        mesh=plsc.VectorSubcoreMesh(**_SC_MESH),
        compiler_params=pltpu.CompilerParams(
            needs_layout_passes=False, use_tc_tiling_on_sc=True),
        scratch_types=[
            pltpu.VMEM((BPW,), jnp.int32),
            pltpu.VMEM((BPW,), jnp.int32),
            pltpu.VMEM((BPW,), jnp.int32),
            pltpu.VMEM((BPW,), jnp.int32),
            pltpu.VMEM((2, QTR, 2 * D), jnp.int32),
            pltpu.VMEM((2, QTR, 2 * D), jnp.int32),
            pltpu.VMEM((BPW,), jnp.float32),
            pltpu.VMEM((BPW,), jnp.float32),
            pltpu.SemaphoreType.DMA,
            pltpu.SemaphoreType.DMA,
        ],
    )(_glove_sc)
    out = fn(c, s, merged, bsum)
    return out.reshape(B, 1)


def kernel(c, s, c_weight, c_biase, s_weight, s_biase):
    return _glove(c.astype(jnp.int32), s.astype(jnp.int32),
                  c_weight.T, c_biase.reshape(VOCAB),
                  s_weight.T, s_biase.reshape(VOCAB))


# speedup vs baseline: 1.0004x; 1.0004x over previous
import functools

import jax
import jax.numpy as jnp
from jax import lax
from jax.experimental import pallas as pl
from jax.experimental.pallas import tpu as pltpu
from jax.experimental.pallas import tpu_sc as plsc

VOCAB = 1000000
D = 64
B = 16384

NC = 2
NS = 16
NW = NC * NS
BPW = B // NW
QTR = BPW // 4
L = 16
UNROLL = 16

CV = 32768
GRID = (VOCAB + CV - 1) // CV
CVLOG = CV.bit_length() - 1

_SC_MESH = dict(core_axis_name="c", subcore_axis_name="s")


def _pack_tc(cwt_ref, swt_ref, p_ref):
    x = jnp.concatenate([cwt_ref[...], swt_ref[...]], axis=0)
    y = jax.lax.dot_general(
        x, jnp.eye(2 * D, dtype=jnp.float32), (((0,), (0,)), ((), ())),
        preferred_element_type=jnp.float32)
    yi = jax.lax.bitcast_convert_type(y, jnp.int32)
    rnd = jnp.int32(0x8000)
    hi = (yi[: CV // 2] + rnd) & jnp.int32(-65536)
    lo = jax.lax.shift_right_logical(yi[CV // 2:] + rnd, 16)
    p_ref[...] = hi | lo


def _pack_tables(cwt, swt):
    return pl.pallas_call(
        _pack_tc,
        grid=(GRID,),
        compiler_params=pltpu.CompilerParams(
            fuse_transposed_lhs_in_matmul=True),
        in_specs=[
            pl.BlockSpec((D, CV), lambda i: (0, i)),
            pl.BlockSpec((D, CV), lambda i: (0, i)),
        ],
        out_specs=pl.BlockSpec((CV // 2, 2 * D), lambda i: (i, 0)),
        out_shape=jax.ShapeDtypeStruct((GRID * CV // 2, 2 * D), jnp.int32),
    )(cwt, swt)


def _bias_sc(c_hbm, s_hbm, cb_hbm, sb_hbm, out_hbm,
             cidx_v, sidx_v, cb_v, sb_v, out_v, sem):
    wid = lax.axis_index("s") * NC + lax.axis_index("c")
    base = wid * BPW

    pltpu.sync_copy(c_hbm.at[pl.ds(base, BPW)], cidx_v)
    pltpu.sync_copy(s_hbm.at[pl.ds(base, BPW)], sidx_v)
    cp1 = pltpu.async_copy(cb_hbm.at[cidx_v], cb_v, sem)
    cp2 = pltpu.async_copy(sb_hbm.at[sidx_v], sb_v, sem)
    cp1.wait()
    cp2.wait()

    def body(g, carry):
        gs = g * L
        out_v[pl.ds(gs, L)] = cb_v[pl.ds(gs, L)] + sb_v[pl.ds(gs, L)]
        return carry

    lax.fori_loop(0, BPW // L, body, 0)
    pltpu.sync_copy(out_v, out_hbm.at[pl.ds(base, BPW)])


def _bias_sums(c, s, cb, sb):
    fn = functools.partial(
        pl.kernel,
        out_type=jax.ShapeDtypeStruct((B,), jnp.float32),
        mesh=plsc.VectorSubcoreMesh(**_SC_MESH),
        compiler_params=pltpu.CompilerParams(
            needs_layout_passes=False, use_tc_tiling_on_sc=False),
        scratch_types=[
            pltpu.VMEM((BPW,), jnp.int32),
            pltpu.VMEM((BPW,), jnp.int32),
            pltpu.VMEM((BPW,), jnp.float32),
            pltpu.VMEM((BPW,), jnp.float32),
            pltpu.VMEM((BPW,), jnp.float32),
            pltpu.SemaphoreType.DMA,
        ],
    )(_bias_sc)
    return fn(c, s, cb, sb)


def _glove_sc(c_hbm, s_hbm, m_hbm, bsum_hbm, out_hbm,
              cidx_v, sidx_v, cpair_v, spair_v, cbuf_v, sbuf_v, bsum_v,
              out_v, sem0, sem1):
    wid = lax.axis_index("s") * NC + lax.axis_index("c")
    base = wid * BPW

    pltpu.sync_copy(c_hbm.at[pl.ds(base, BPW)], cidx_v)
    pltpu.sync_copy(s_hbm.at[pl.ds(base, BPW)], sidx_v)
    pltpu.sync_copy(bsum_hbm.at[pl.ds(base, BPW)], bsum_v)

    def pair(v):
        return (lax.shift_left(lax.shift_right_logical(v, CVLOG), CVLOG - 1)
                | (v & (CV // 2 - 1)))

    def pair_body(g, carry):
        gs = g * L
        cpair_v[pl.ds(gs, L)] = pair(cidx_v[pl.ds(gs, L)])
        spair_v[pl.ds(gs, L)] = pair(sidx_v[pl.ds(gs, L)])
        return carry

    lax.fori_loop(0, BPW // L, pair_body, 0)

    nq = BPW // QTR

    def fire(q, buf_slot):
        qs = q * QTR
        sem = sem0 if buf_slot == 0 else sem1
        return (
            pltpu.async_copy(
                m_hbm.at[cpair_v.at[pl.ds(qs, QTR)]], cbuf_v.at[buf_slot],
                sem),
            pltpu.async_copy(
                m_hbm.at[spair_v.at[pl.ds(qs, QTR)]], sbuf_v.at[buf_slot],
                sem),
        )

    cps = {0: fire(0, 0)}
    for q in range(nq):
        if q + 1 < nq:
            cps[q + 1] = fire(q + 1, (q + 1) % 2)
        cp1, cp2 = cps.pop(q)
        cp1.wait()
        cp2.wait()
        qs = q * QTR
        slot = q % 2

        def group_body(g, carry2, qs=qs, slot=slot):
            gs = g * L
            row_idx = gs + lax.iota(jnp.int32, L)
            csh = (lax.shift_right_logical(
                cidx_v[pl.ds(qs + gs, L)], CVLOG - 1) & 1) << 4
            ssh = (lax.shift_right_logical(
                sidx_v[pl.ds(qs + gs, L)], CVLOG - 1) & 1) << 4
            mask = jnp.full((L,), -65536, jnp.int32)
            acc = bsum_v[pl.ds(qs + gs, L)]

            def d_body(d0, acc):
                for u in range(UNROLL):
                    d = d0 * UNROLL + u
                    col_c = jnp.full((L,), 0, jnp.int32) + d
                    col_s = col_c + D
                    wc = plsc.load_gather(cbuf_v.at[slot], [row_idx, col_c])
                    ws = plsc.load_gather(sbuf_v.at[slot], [row_idx, col_s])
                    cv = plsc.bitcast((wc << csh) & mask, jnp.float32)
                    sv = plsc.bitcast((ws << ssh) & mask, jnp.float32)
                    acc = acc + cv * sv
                return acc

            acc = lax.fori_loop(0, D // UNROLL, d_body, acc)
            out_v[pl.ds(qs + gs, L)] = acc
            return carry2

        lax.fori_loop(0, QTR // L, group_body, 0)

    pltpu.sync_copy(out_v, out_hbm.at[pl.ds(base, BPW)])


@jax.jit
def _glove(c, s, cwt, cb, swt, sb):
    bsum = _bias_sums(c, s, cb, sb)
    merged = _pack_tables(cwt, swt)
    fn = functools.partial(
        pl.kernel,
        out_type=jax.ShapeDtypeStruct((B,), jnp.float32),
        mesh=plsc.VectorSubcoreMesh(**_SC_MESH),
        compiler_params=pltpu.CompilerParams(
            needs_layout_passes=False, use_tc_tiling_on_sc=True),
        scratch_types=[
            pltpu.VMEM((BPW,), jnp.int32),
            pltpu.VMEM((BPW,), jnp.int32),
            pltpu.VMEM((BPW,), jnp.int32),
            pltpu.VMEM((BPW,), jnp.int32),
            pltpu.VMEM((2, QTR, 2 * D), jnp.int32),
            pltpu.VMEM((2, QTR, 2 * D), jnp.int32),
            pltpu.VMEM((BPW,), jnp.float32),
            pltpu.VMEM((BPW,), jnp.float32),
            pltpu.SemaphoreType.DMA,
            pltpu.SemaphoreType.DMA,
        ],
    )(_glove_sc)
    out = fn(c, s, merged, bsum)
    return out.reshape(B, 1)


def kernel(c, s, c_weight, c_biase, s_weight, s_biase):
    return _glove(c.astype(jnp.int32), s.astype(jnp.int32),
                  c_weight.T, c_biase.reshape(VOCAB),
                  s_weight.T, s_biase.reshape(VOCAB))


# speedup vs baseline: 1.0007x; 1.0003x over previous
import functools

import jax
import jax.numpy as jnp
from jax import lax
from jax.experimental import pallas as pl
from jax.experimental.pallas import tpu as pltpu
from jax.experimental.pallas import tpu_sc as plsc

VOCAB = 1000000
D = 64
B = 16384

NC = 2
NS = 16
NW = NC * NS
BPW = B // NW
QTR = BPW // 4
L = 16
UNROLL = 8

CV = 32768
GRID = (VOCAB + CV - 1) // CV
CVLOG = CV.bit_length() - 1

_SC_MESH = dict(core_axis_name="c", subcore_axis_name="s")


def _pack_tc(cwt_ref, swt_ref, p_ref):
    x = jnp.concatenate([cwt_ref[...], swt_ref[...]], axis=0)
    y = jax.lax.dot_general(
        x, jnp.eye(2 * D, dtype=jnp.float32), (((0,), (0,)), ((), ())),
        preferred_element_type=jnp.float32)
    yi = jax.lax.bitcast_convert_type(y, jnp.int32)
    rnd = jnp.int32(0x8000)
    hi = (yi[: CV // 2] + rnd) & jnp.int32(-65536)
    lo = jax.lax.shift_right_logical(yi[CV // 2:] + rnd, 16)
    p_ref[...] = hi | lo


def _pack_tables(cwt, swt):
    return pl.pallas_call(
        _pack_tc,
        grid=(GRID,),
        compiler_params=pltpu.CompilerParams(
            fuse_transposed_lhs_in_matmul=True),
        in_specs=[
            pl.BlockSpec((D, CV), lambda i: (0, i)),
            pl.BlockSpec((D, CV), lambda i: (0, i)),
        ],
        out_specs=pl.BlockSpec((CV // 2, 2 * D), lambda i: (i, 0)),
        out_shape=jax.ShapeDtypeStruct((GRID * CV // 2, 2 * D), jnp.int32),
    )(cwt, swt)


def _bias_sc(c_hbm, s_hbm, cb_hbm, sb_hbm, out_hbm,
             cidx_v, sidx_v, cb_v, sb_v, out_v, sem):
    wid = lax.axis_index("s") * NC + lax.axis_index("c")
    base = wid * BPW

    pltpu.sync_copy(c_hbm.at[pl.ds(base, BPW)], cidx_v)
    pltpu.sync_copy(s_hbm.at[pl.ds(base, BPW)], sidx_v)
    cp1 = pltpu.async_copy(cb_hbm.at[cidx_v], cb_v, sem)
    cp2 = pltpu.async_copy(sb_hbm.at[sidx_v], sb_v, sem)
    cp1.wait()
    cp2.wait()

    def body(g, carry):
        gs = g * L
        out_v[pl.ds(gs, L)] = cb_v[pl.ds(gs, L)] + sb_v[pl.ds(gs, L)]
        return carry

    lax.fori_loop(0, BPW // L, body, 0)
    pltpu.sync_copy(out_v, out_hbm.at[pl.ds(base, BPW)])


def _bias_sums(c, s, cb, sb):
    fn = functools.partial(
        pl.kernel,
        out_type=jax.ShapeDtypeStruct((B,), jnp.float32),
        mesh=plsc.VectorSubcoreMesh(**_SC_MESH),
        compiler_params=pltpu.CompilerParams(
            needs_layout_passes=False, use_tc_tiling_on_sc=False),
        scratch_types=[
            pltpu.VMEM((BPW,), jnp.int32),
            pltpu.VMEM((BPW,), jnp.int32),
            pltpu.VMEM((BPW,), jnp.float32),
            pltpu.VMEM((BPW,), jnp.float32),
            pltpu.VMEM((BPW,), jnp.float32),
            pltpu.SemaphoreType.DMA,
        ],
    )(_bias_sc)
    return fn(c, s, cb, sb)


def _glove_sc(c_hbm, s_hbm, m_hbm, bsum_hbm, out_hbm,
              cidx_v, sidx_v, cpair_v, spair_v, cbuf_v, sbuf_v, bsum_v,
              out_v, sem0, sem1):
    wid = lax.axis_index("s") * NC + lax.axis_index("c")
    base = wid * BPW

    pltpu.sync_copy(c_hbm.at[pl.ds(base, BPW)], cidx_v)
    pltpu.sync_copy(s_hbm.at[pl.ds(base, BPW)], sidx_v)
    pltpu.sync_copy(bsum_hbm.at[pl.ds(base, BPW)], bsum_v)

    def pair(v):
        return (lax.shift_left(lax.shift_right_logical(v, CVLOG), CVLOG - 1)
                | (v & (CV // 2 - 1)))

    def pair_body(g, carry):
        gs = g * L
        cpair_v[pl.ds(gs, L)] = pair(cidx_v[pl.ds(gs, L)])
        spair_v[pl.ds(gs, L)] = pair(sidx_v[pl.ds(gs, L)])
        return carry

    lax.fori_loop(0, BPW // L, pair_body, 0)

    nq = BPW // QTR

    def fire(q, buf_slot):
        qs = q * QTR
        sem = sem0 if buf_slot == 0 else sem1
        return (
            pltpu.async_copy(
                m_hbm.at[cpair_v.at[pl.ds(qs, QTR)]], cbuf_v.at[buf_slot],
                sem),
            pltpu.async_copy(
                m_hbm.at[spair_v.at[pl.ds(qs, QTR)]], sbuf_v.at[buf_slot],
                sem),
        )

    cps = {0: fire(0, 0)}
    for q in range(nq):
        if q + 1 < nq:
            cps[q + 1] = fire(q + 1, (q + 1) % 2)
        cp1, cp2 = cps.pop(q)
        cp1.wait()
        cp2.wait()
        qs = q * QTR
        slot = q % 2

        def group_body(g, carry2, qs=qs, slot=slot):
            gs = g * L
            row_idx = gs + lax.iota(jnp.int32, L)
            csh = (lax.shift_right_logical(
                cidx_v[pl.ds(qs + gs, L)], CVLOG - 1) & 1) << 4
            ssh = (lax.shift_right_logical(
                sidx_v[pl.ds(qs + gs, L)], CVLOG - 1) & 1) << 4
            mask = jnp.full((L,), -65536, jnp.int32)
            acc = bsum_v[pl.ds(qs + gs, L)]

            def d_body(d0, acc):
                for u in range(UNROLL):
                    d = d0 * UNROLL + u
                    col_c = jnp.full((L,), 0, jnp.int32) + d
                    col_s = col_c + D
                    wc = plsc.load_gather(cbuf_v.at[slot], [row_idx, col_c])
                    ws = plsc.load_gather(sbuf_v.at[slot], [row_idx, col_s])
                    cv = plsc.bitcast((wc << csh) & mask, jnp.float32)
                    sv = plsc.bitcast((ws << ssh) & mask, jnp.float32)
                    acc = acc + cv * sv
                return acc

            acc = lax.fori_loop(0, D // UNROLL, d_body, acc)
            out_v[pl.ds(qs + gs, L)] = acc
            return carry2

        lax.fori_loop(0, QTR // L, group_body, 0)

    pltpu.sync_copy(out_v, out_hbm.at[pl.ds(base, BPW)])


@jax.jit
def _glove(c, s, cwt, cb, swt, sb):
    bsum = _bias_sums(c, s, cb, sb)
    merged = _pack_tables(cwt, swt)
    fn = functools.partial(
        pl.kernel,
        out_type=jax.ShapeDtypeStruct((B,), jnp.float32),
        mesh=plsc.VectorSubcoreMesh(**_SC_MESH),
        compiler_params=pltpu.CompilerParams(
            needs_layout_passes=False, use_tc_tiling_on_sc=True),
        scratch_types=[
            pltpu.VMEM((BPW,), jnp.int32),
            pltpu.VMEM((BPW,), jnp.int32),
            pltpu.VMEM((BPW,), jnp.int32),
            pltpu.VMEM((BPW,), jnp.int32),
            pltpu.VMEM((2, QTR, 2 * D), jnp.int32),
            pltpu.VMEM((2, QTR, 2 * D), jnp.int32),
            pltpu.VMEM((BPW,), jnp.float32),
            pltpu.VMEM((BPW,), jnp.float32),
            pltpu.SemaphoreType.DMA,
            pltpu.SemaphoreType.DMA,
        ],
    )(_glove_sc)
    out = fn(c, s, merged, bsum)
    return out.reshape(B, 1)


def kernel(c, s, c_weight, c_biase, s_weight, s_biase):
    return _glove(c.astype(jnp.int32), s.astype(jnp.int32),
                  c_weight.T, c_biase.reshape(VOCAB),
                  s_weight.T, s_biase.reshape(VOCAB))
